# baseline (speedup 1.0000x reference)
import functools

import jax
import jax.numpy as jnp
from jax.experimental import pallas as pl
from jax.experimental.pallas import tpu as pltpu

_K_TOP = 256
_N = 32768
_EPS = 0.1
_MAX_ITER = 200
_PAD = 1e-16
_BC = 16


def _body(s_ref, out_ref, m_ref, s_scr, t_ref):
    p = pl.program_id(0)
    i = pl.program_id(1)
    n = s_ref.shape[1]
    mu = 1.0 / n
    nu0 = _K_TOP / n
    nu1 = (n - _K_TOP) / n

    @pl.when(p == 0)
    def _phase_max():
        s = s_ref[...]
        s_scr[pl.ds(i * _BC, _BC), :] = s
        local = jnp.max(jnp.maximum(s * s, (s - 1.0) * (s - 1.0)))
        @pl.when(i == 0)
        def _():
            m_ref[0] = local
        @pl.when(i != 0)
        def _():
            m_ref[0] = jnp.maximum(m_ref[0], local)

    @pl.when(p == 1)
    def _phase_sinkhorn():
        s = s_scr[pl.ds(i * _BC, _BC), :]
        c = (1.0 / _EPS) / m_ref[0]
        sc = s * c
        t = jnp.exp(2.0 * sc - c)
        g0 = jnp.exp(-(sc * s))
        t_ref[...] = t

        r0 = jnp.sum(g0, axis=1, keepdims=True)
        r1 = jnp.sum(g0 * t, axis=1, keepdims=True)
        u0 = nu0 / (r0 + _PAD)
        u1 = nu1 / (r1 + _PAD)

        def cond(carry):
            it, _, _, changed = carry
            return jnp.logical_and(it < _MAX_ITER - 1, changed)

        def body(carry):
            it, u0, u1, _ = carry
            w = pl.reciprocal(u0 + u1 * t_ref[...], approx=True)
            s0 = jnp.sum(w, axis=1, keepdims=True)
            r0 = mu * s0
            r1 = mu * (n - u0 * s0) / u1
            n0 = nu0 / (r0 + _PAD)
            n1 = nu1 / (r1 + _PAD)
            changed = jnp.logical_or(
                jnp.any(jnp.abs(n0 - u0) > 1e-6 * u0),
                jnp.any(jnp.abs(n1 - u1) > 1e-6 * u1),
            )
            return it + 1, n0, n1, changed

        _, u0, u1, _ = jax.lax.while_loop(
            cond, body, (jnp.int32(0), u0, u1, jnp.bool_(True))
        )

        tt = t_ref[...]
        w = 1.0 / (u0 + u1 * tt)
        out_ref[:, :n] = (mu * u0) * w
        out_ref[:, n:] = (mu * u1) * (tt * w)


@functools.partial(jax.jit, static_argnames=())
def kernel(scores):
    b, n = scores.shape
    g = b // _BC
    out = pl.pallas_call(
        _body,
        grid=(2, g),
        in_specs=[
            pl.BlockSpec((_BC, n), lambda p, i: ((1 - p) * i + p * (g - 1), 0)),
        ],
        out_specs=pl.BlockSpec((_BC, 2 * n), lambda p, i: (p * i, 0)),
        out_shape=jax.ShapeDtypeStruct((b, 2 * n), jnp.float32),
        scratch_shapes=[
            pltpu.SMEM((1,), jnp.float32),
            pltpu.VMEM((b, n), jnp.float32),
            pltpu.VMEM((_BC, n), jnp.float32),
        ],
    )(scores)
    return out.reshape(b, 2, n)
